# Initial kernel scaffold; baseline (speedup 1.0000x reference)
import jax
import jax.numpy as jnp
from jax.experimental import pallas as pl


def kernel(x, edge_index, edge_attr, y, params):
    raise NotImplementedError("write your pallas kernel here")


# speedup vs baseline: 3.7080x; 3.7080x over previous
import functools

import jax
import jax.numpy as jnp
from jax import lax
from jax.experimental import pallas as pl
from jax.experimental.pallas import tpu as pltpu
from jax.experimental.pallas import tpu_sc as plsc

N_NODES = 10000
N_EDGES = 320000
N_ENC = 128
E_ENC = 16
MD = 64
NUM_LAYERS = 3
NUM_GRAPHS = 100
NODES_PER_GRAPH = 100

NC = 2
NS = 16
NW = NC * NS
EDGES_PER_W = N_EDGES // NW
K = 80
NCHUNK = EDGES_PER_W // K
G = 25
NGRP = NCHUNK // G
N_PAD = 10240
RPS = N_PAD // NS


def _mm_bias_body(a_ref, w_ref, b_ref, o_ref):
    o_ref[...] = (
        jnp.dot(a_ref[...], w_ref[...], preferred_element_type=jnp.float32)
        + b_ref[...]
    )


def _mm_bias_pad_body(a_ref, w_ref, b_ref, o_ref):
    r = (jnp.dot(a_ref[...], w_ref[...], preferred_element_type=jnp.float32)
         + b_ref[...])
    o_ref[...] = jnp.concatenate([r, jnp.zeros_like(r)], axis=1)


def _mm_bias(a, w, b, block_rows, out_rows=None, pad_cols=False):
    rows, kdim = a.shape
    n = w.shape[1]
    out_n = 2 * n if pad_cols else n
    return pl.pallas_call(
        _mm_bias_pad_body if pad_cols else _mm_bias_body,
        grid=(rows // block_rows,),
        in_specs=[
            pl.BlockSpec((block_rows, kdim), lambda i: (i, 0)),
            pl.BlockSpec((kdim, n), lambda i: (0, 0)),
            pl.BlockSpec((1, n), lambda i: (0, 0)),
        ],
        out_specs=pl.BlockSpec((block_rows, out_n), lambda i: (i, 0)),
        out_shape=jax.ShapeDtypeStruct((out_rows or rows, out_n), jnp.float32),
    )(a, w, b.reshape(1, n))


def _eall_body(eap_ref, ew_ref, mw_ref, eb_ref, mb_ref, o0_ref, o1_ref, o2_ref):
    w2 = jnp.dot(ew_ref[...], mw_ref[...], preferred_element_type=jnp.float32)
    d = jnp.dot(eb_ref[...], mw_ref[...], preferred_element_type=jnp.float32) + mb_ref[...]
    ae = eap_ref[:, :E_ENC]
    ao = eap_ref[:, E_ENC:]
    ee = jnp.dot(ae, w2, preferred_element_type=jnp.float32) + d
    eo = jnp.dot(ao, w2, preferred_element_type=jnp.float32) + d
    for i, o_ref in enumerate((o0_ref, o1_ref, o2_ref)):
        o_ref[...] = jnp.concatenate(
            [ee[:, i * MD:(i + 1) * MD], eo[:, i * MD:(i + 1) * MD]], axis=1)


def _edge_terms(ea_pair, ew, mwcat, eb, mbcat, block_rows=2000):
    n_pair = N_EDGES // 2
    spec = pl.BlockSpec((block_rows, 2 * MD), lambda i: (i, 0))
    return pl.pallas_call(
        _eall_body,
        grid=(n_pair // block_rows,),
        in_specs=[
            pl.BlockSpec((block_rows, 2 * E_ENC), lambda i: (i, 0)),
            pl.BlockSpec((E_ENC, MD), lambda i: (0, 0)),
            pl.BlockSpec((MD, 3 * MD), lambda i: (0, 0)),
            pl.BlockSpec((1, MD), lambda i: (0, 0)),
            pl.BlockSpec((1, 3 * MD), lambda i: (0, 0)),
        ],
        out_specs=(spec, spec, spec),
        out_shape=tuple(
            jax.ShapeDtypeStruct((n_pair, 2 * MD), jnp.float32) for _ in range(3)
        ),
    )(ea_pair, ew, mwcat, eb.reshape(1, MD), mbcat.reshape(1, 3 * MD))


def _upd_body(nf_ref, a0_ref, a1_ref, wt_ref, wb_ref, b_ref, mw_ref, nfo_ref, nfwo_ref):
    agg = a0_ref[:, :MD] + a1_ref[:, :MD]
    h = (
        jnp.dot(nf_ref[...], wt_ref[...], preferred_element_type=jnp.float32)
        + jnp.dot(agg, wb_ref[...], preferred_element_type=jnp.float32)
        + b_ref[...]
    )
    nfn = nf_ref[...] + jnp.maximum(h, 0.0)
    nfo_ref[...] = nfn
    r = jnp.dot(nfn, mw_ref[...], preferred_element_type=jnp.float32)
    nfwo_ref[...] = jnp.concatenate([r, jnp.zeros_like(r)], axis=1)


def _upd_last_body(nf_ref, a0_ref, a1_ref, wt_ref, wb_ref, b_ref, nfo_ref):
    agg = a0_ref[:, :MD] + a1_ref[:, :MD]
    h = (
        jnp.dot(nf_ref[...], wt_ref[...], preferred_element_type=jnp.float32)
        + jnp.dot(agg, wb_ref[...], preferred_element_type=jnp.float32)
        + b_ref[...]
    )
    nfo_ref[...] = nf_ref[...] + jnp.maximum(h, 0.0)


def _update(nf, a0, a1, wt, wb, b, mw_next, block_rows=2000):
    nspec = pl.BlockSpec((block_rows, MD), lambda i: (i, 0))
    aspec = pl.BlockSpec((block_rows, 2 * MD), lambda i: (i, 0))
    wspec = pl.BlockSpec((MD, MD), lambda i: (0, 0))
    bspec = pl.BlockSpec((1, MD), lambda i: (0, 0))
    grid = (N_NODES // block_rows,)
    if mw_next is None:
        return pl.pallas_call(
            _upd_last_body,
            grid=grid,
            in_specs=[nspec, aspec, aspec, wspec, wspec, bspec],
            out_specs=nspec,
            out_shape=jax.ShapeDtypeStruct((N_NODES, MD), jnp.float32),
        )(nf, a0, a1, wt, wb, b.reshape(1, MD))
    return pl.pallas_call(
        _upd_body,
        grid=grid,
        in_specs=[nspec, aspec, aspec, wspec, wspec, bspec, wspec],
        out_specs=(nspec, pl.BlockSpec((block_rows, 2 * MD), lambda i: (i, 0))),
        out_shape=(
            jax.ShapeDtypeStruct((N_NODES, MD), jnp.float32),
            jax.ShapeDtypeStruct((N_PAD, 2 * MD), jnp.float32),
        ),
    )(nf, a0, a1, wt, wb, b.reshape(1, MD), mw_next)


def _head_body(h_ref, y_ref, w0_ref, b0_ref, w1_ref, b1_ref, w2_ref, b2_ref, o_ref):
    h1 = jnp.maximum(
        jnp.dot(h_ref[...], w0_ref[...], preferred_element_type=jnp.float32) + b0_ref[...], 0.0)
    h2 = jnp.maximum(
        jnp.dot(h1, w1_ref[...], preferred_element_type=jnp.float32) + b1_ref[...], 0.0)
    yh = jnp.dot(h2, w2_ref[...], preferred_element_type=jnp.float32) + b2_ref[...]
    d = yh - y_ref[...]
    valid = lax.broadcasted_iota(jnp.int32, d.shape, 0) < 5 * NUM_GRAPHS
    d = jnp.where(valid, d, 0.0)
    o_ref[...] = (jnp.sum(d * d) / (5.0 * NUM_GRAPHS)).reshape(1, 1)


def _head(h_pad, y_pad, p):
    rows = h_pad.shape[0]
    return pl.pallas_call(
        _head_body,
        in_specs=[
            pl.BlockSpec((rows, 2 * MD), lambda: (0, 0)),
            pl.BlockSpec((rows, 1), lambda: (0, 0)),
            pl.BlockSpec((2 * MD, MD), lambda: (0, 0)),
            pl.BlockSpec((1, MD), lambda: (0, 0)),
            pl.BlockSpec((MD, MD // 2), lambda: (0, 0)),
            pl.BlockSpec((1, MD // 2), lambda: (0, 0)),
            pl.BlockSpec((MD // 2, 1), lambda: (0, 0)),
            pl.BlockSpec((1, 1), lambda: (0, 0)),
        ],
        out_specs=pl.BlockSpec((1, 1), lambda: (0, 0)),
        out_shape=jax.ShapeDtypeStruct((1, 1), jnp.float32),
    )(
        h_pad, y_pad,
        p['mlp_w0'], p['mlp_b0'].reshape(1, MD),
        p['mlp_w1'], p['mlp_b1'].reshape(1, MD // 2),
        p['mlp_w2'], p['mlp_b2'].reshape(1, 1),
    )


@functools.lru_cache(maxsize=1)
def _make_edge_pass():
    mesh = plsc.VectorSubcoreMesh(core_axis_name="c", subcore_axis_name="s")

    @functools.partial(
        pl.kernel,
        out_type=jax.ShapeDtypeStruct((2, N_PAD, 2 * MD), jnp.float32),
        mesh=mesh,
        scratch_types=[
            pltpu.VMEM((G, K), jnp.int32),
            pltpu.VMEM((G, K), jnp.int32),
            pltpu.VMEM((2, K, 2 * MD), jnp.float32),
            pltpu.VMEM((2, K // 2, 2 * MD), jnp.float32),
            pltpu.VMEM_SHARED((N_PAD, 2 * MD), jnp.float32),
            pltpu.SemaphoreType.DMA,
            pltpu.SemaphoreType.DMA,
            pltpu.SemaphoreType.DMA,
        ],
    )
    def _edge_pass_k(nfw_hbm, e_hbm, src_hbm, dst_hbm, zeros_hbm, out_hbm,
                     src_v, dst_v, g_v, e_v, agg_sh, gsem, esem, ssem):
        c = lax.axis_index("c")
        s = lax.axis_index("s")
        wid = s * NC + c

        row0 = pl.multiple_of(s * RPS, 8)
        pltpu.sync_copy(zeros_hbm.at[pl.ds(row0, RPS)],
                        agg_sh.at[pl.ds(row0, RPS)])
        plsc.subcore_barrier()

        def _issue(grp, cg, b):
            pltpu.async_copy(nfw_hbm.at[src_v.at[cg]], g_v.at[b], gsem)
            pltpu.async_copy(e_hbm.at[wid, grp, cg], e_v.at[b], esem)

        def _wait_in(grp, cg, b):
            pltpu.make_async_copy(
                nfw_hbm.at[src_v.at[cg]], g_v.at[b], gsem).wait()
            pltpu.make_async_copy(
                e_hbm.at[wid, grp, cg], e_v.at[b], esem).wait()

        def _wait_scat(cg, b):
            pltpu.make_async_copy(
                g_v.at[b], agg_sh.at[dst_v.at[cg]], ssem).wait()

        def _step(grp, cg, b):
            @pl.when(cg >= 1)
            def _():
                _wait_scat(cg - 1, 1 - b)

            @pl.when(cg + 1 < G)
            def _():
                _issue(grp, cg + 1, 1 - b)

            _wait_in(grp, cg, b)

            @pl.loop(0, K // 2)
            def _pair(prow):
                for half in range(2):
                    r = 2 * prow + half
                    for cc in range(MD // 16):
                        g_v[b, r, pl.ds(cc * 16, 16)] = jnp.maximum(
                            g_v[b, r, pl.ds(cc * 16, 16)]
                            + e_v[b, prow, pl.ds(half * MD + cc * 16, 16)],
                            0.0)

            pltpu.async_copy(g_v.at[b], agg_sh.at[dst_v.at[cg]], ssem, add=True)

        for grp in range(NGRP):
            pltpu.sync_copy(src_hbm.at[wid, grp], src_v)
            pltpu.sync_copy(dst_hbm.at[wid, grp], dst_v)
            _issue(grp, 0, 0)

            @pl.loop(0, G - 1, step=2)
            def _chunk(cg0):
                _step(grp, cg0, 0)
                _step(grp, cg0 + 1, 1)

            _step(grp, G - 1, (G - 1) % 2)
            _wait_scat(G - 1, (G - 1) % 2)

        plsc.subcore_barrier()
        pltpu.sync_copy(agg_sh.at[pl.ds(row0, RPS)],
                        out_hbm.at[c, pl.ds(row0, RPS)])

    return _edge_pass_k


def kernel(x, edge_index, edge_attr, y, params):
    p = params
    ei = edge_index.astype(jnp.int32)
    src3 = ei[0].reshape(NW, NGRP, G, K)
    dst3 = ei[1].reshape(NW, NGRP, G, K)
    zeros = jnp.zeros((N_PAD, 2 * MD), jnp.float32)
    ea_pair = edge_attr.reshape(N_EDGES // 2, 2 * E_ENC)

    mwcat = jnp.concatenate([p['msg_w_%d' % i] for i in range(NUM_LAYERS)], axis=1)
    mbcat = jnp.concatenate([p['msg_b_%d' % i] for i in range(NUM_LAYERS)])

    nf = _mm_bias(x, p['node_enc_w'], p['node_enc_b'], block_rows=2000)
    e_terms = _edge_terms(ea_pair, p['edge_enc_w'], mwcat, p['edge_enc_b'], mbcat)
    e_terms = [e.reshape(NW, NGRP, G, K // 2, 2 * MD) for e in e_terms]

    zeros_b = jnp.zeros((MD,), jnp.float32)
    nfw = _mm_bias(nf, p['msg_w_0'], zeros_b, block_rows=2000,
                   out_rows=N_PAD, pad_cols=True)

    for i in range(NUM_LAYERS):
        agg2 = _make_edge_pass()(nfw, e_terms[i], src3, dst3, zeros)
        wt = p['upd_w_%d' % i][:MD]
        wb = p['upd_w_%d' % i][MD:]
        mw_next = p['msg_w_%d' % (i + 1)] if i + 1 < NUM_LAYERS else None
        res = _update(nf, agg2[0], agg2[1], wt, wb, p['upd_b_%d' % i], mw_next)
        if mw_next is None:
            nf = res
        else:
            nf, nfw = res

    nfr = nf.reshape(NUM_GRAPHS, NODES_PER_GRAPH, MD)
    h = jnp.concatenate([nfr[:, -10:-5, :], nfr[:, -5:, :]], axis=-1)
    h = h.reshape(5 * NUM_GRAPHS, 2 * MD)
    h_pad = jnp.pad(h, ((0, 12), (0, 0)))
    y_pad = jnp.pad(y, (0, 12)).reshape(512, 1)
    loss = _head(h_pad, y_pad, p)
    return loss.reshape(())


# speedup vs baseline: 3.7677x; 1.0161x over previous
import functools

import jax
import jax.numpy as jnp
from jax import lax
from jax.experimental import pallas as pl
from jax.experimental.pallas import tpu as pltpu
from jax.experimental.pallas import tpu_sc as plsc

N_NODES = 10000
N_EDGES = 320000
N_ENC = 128
E_ENC = 16
MD = 64
NUM_LAYERS = 3
NUM_GRAPHS = 100
NODES_PER_GRAPH = 100

NC = 2
NS = 16
NW = NC * NS
EDGES_PER_W = N_EDGES // NW
K = 80
NCHUNK = EDGES_PER_W // K
G = 25
NGRP = NCHUNK // G
N_PAD = 10240
RPS = N_PAD // NS


def _mm_bias_body(a_ref, w_ref, b_ref, o_ref):
    o_ref[...] = (
        jnp.dot(a_ref[...], w_ref[...], preferred_element_type=jnp.float32)
        + b_ref[...]
    )


def _mm_bias_pad_body(a_ref, w_ref, b_ref, o_ref):
    r = (jnp.dot(a_ref[...], w_ref[...], preferred_element_type=jnp.float32)
         + b_ref[...])
    o_ref[...] = jnp.concatenate([r, jnp.zeros_like(r)], axis=1)


def _mm_bias(a, w, b, block_rows, out_rows=None, pad_cols=False):
    rows, kdim = a.shape
    n = w.shape[1]
    out_n = 2 * n if pad_cols else n
    return pl.pallas_call(
        _mm_bias_pad_body if pad_cols else _mm_bias_body,
        grid=(rows // block_rows,),
        in_specs=[
            pl.BlockSpec((block_rows, kdim), lambda i: (i, 0)),
            pl.BlockSpec((kdim, n), lambda i: (0, 0)),
            pl.BlockSpec((1, n), lambda i: (0, 0)),
        ],
        out_specs=pl.BlockSpec((block_rows, out_n), lambda i: (i, 0)),
        out_shape=jax.ShapeDtypeStruct((out_rows or rows, out_n), jnp.float32),
    )(a, w, b.reshape(1, n))


def _eall_body(eap_ref, ew_ref, mw_ref, eb_ref, mb_ref, o0_ref, o1_ref, o2_ref):
    w2 = jnp.dot(ew_ref[...], mw_ref[...], preferred_element_type=jnp.float32)
    d = jnp.dot(eb_ref[...], mw_ref[...], preferred_element_type=jnp.float32) + mb_ref[...]
    ae = eap_ref[:, :E_ENC]
    ao = eap_ref[:, E_ENC:]
    ee = jnp.dot(ae, w2, preferred_element_type=jnp.float32) + d
    eo = jnp.dot(ao, w2, preferred_element_type=jnp.float32) + d
    for i, o_ref in enumerate((o0_ref, o1_ref, o2_ref)):
        o_ref[...] = jnp.concatenate(
            [ee[:, i * MD:(i + 1) * MD], eo[:, i * MD:(i + 1) * MD]], axis=1)


def _edge_terms(ea_pair, ew, mwcat, eb, mbcat, block_rows=2000):
    n_pair = N_EDGES // 2
    spec = pl.BlockSpec((block_rows, 2 * MD), lambda i: (i, 0))
    return pl.pallas_call(
        _eall_body,
        grid=(n_pair // block_rows,),
        in_specs=[
            pl.BlockSpec((block_rows, 2 * E_ENC), lambda i: (i, 0)),
            pl.BlockSpec((E_ENC, MD), lambda i: (0, 0)),
            pl.BlockSpec((MD, 3 * MD), lambda i: (0, 0)),
            pl.BlockSpec((1, MD), lambda i: (0, 0)),
            pl.BlockSpec((1, 3 * MD), lambda i: (0, 0)),
        ],
        out_specs=(spec, spec, spec),
        out_shape=tuple(
            jax.ShapeDtypeStruct((n_pair, 2 * MD), jnp.float32) for _ in range(3)
        ),
    )(ea_pair, ew, mwcat, eb.reshape(1, MD), mbcat.reshape(1, 3 * MD))


def _upd_body(nf_ref, a0_ref, a1_ref, wt_ref, wb_ref, b_ref, mw_ref, nfo_ref, nfwo_ref):
    agg = a0_ref[0, :, :MD] + a1_ref[0, :, :MD]
    h = (
        jnp.dot(nf_ref[...], wt_ref[...], preferred_element_type=jnp.float32)
        + jnp.dot(agg, wb_ref[...], preferred_element_type=jnp.float32)
        + b_ref[...]
    )
    nfn = nf_ref[...] + jnp.maximum(h, 0.0)
    nfo_ref[...] = nfn
    r = jnp.dot(nfn, mw_ref[...], preferred_element_type=jnp.float32)
    nfwo_ref[...] = jnp.concatenate([r, jnp.zeros_like(r)], axis=1)


def _upd_last_body(nf_ref, a0_ref, a1_ref, wt_ref, wb_ref, b_ref, nfo_ref):
    agg = a0_ref[0, :, :MD] + a1_ref[0, :, :MD]
    h = (
        jnp.dot(nf_ref[...], wt_ref[...], preferred_element_type=jnp.float32)
        + jnp.dot(agg, wb_ref[...], preferred_element_type=jnp.float32)
        + b_ref[...]
    )
    nfo_ref[...] = nf_ref[...] + jnp.maximum(h, 0.0)


def _update(nf, agg2, wt, wb, b, mw_next, block_rows=2000):
    nspec = pl.BlockSpec((block_rows, MD), lambda i: (i, 0))
    a0spec = pl.BlockSpec((1, block_rows, 2 * MD), lambda i: (0, i, 0))
    a1spec = pl.BlockSpec((1, block_rows, 2 * MD), lambda i: (1, i, 0))
    wspec = pl.BlockSpec((MD, MD), lambda i: (0, 0))
    bspec = pl.BlockSpec((1, MD), lambda i: (0, 0))
    grid = (N_NODES // block_rows,)
    if mw_next is None:
        return pl.pallas_call(
            _upd_last_body,
            grid=grid,
            in_specs=[nspec, a0spec, a1spec, wspec, wspec, bspec],
            out_specs=nspec,
            out_shape=jax.ShapeDtypeStruct((N_NODES, MD), jnp.float32),
        )(nf, agg2, agg2, wt, wb, b.reshape(1, MD))
    return pl.pallas_call(
        _upd_body,
        grid=grid,
        in_specs=[nspec, a0spec, a1spec, wspec, wspec, bspec, wspec],
        out_specs=(nspec, pl.BlockSpec((block_rows, 2 * MD), lambda i: (i, 0))),
        out_shape=(
            jax.ShapeDtypeStruct((N_NODES, MD), jnp.float32),
            jax.ShapeDtypeStruct((N_PAD, 2 * MD), jnp.float32),
        ),
    )(nf, agg2, agg2, wt, wb, b.reshape(1, MD), mw_next)


def _head_body(h_ref, y_ref, w0_ref, b0_ref, w1_ref, b1_ref, w2_ref, b2_ref, o_ref):
    h1 = jnp.maximum(
        jnp.dot(h_ref[...], w0_ref[...], preferred_element_type=jnp.float32) + b0_ref[...], 0.0)
    h2 = jnp.maximum(
        jnp.dot(h1, w1_ref[...], preferred_element_type=jnp.float32) + b1_ref[...], 0.0)
    yh = jnp.dot(h2, w2_ref[...], preferred_element_type=jnp.float32) + b2_ref[...]
    d = yh - y_ref[...]
    valid = lax.broadcasted_iota(jnp.int32, d.shape, 0) < 5 * NUM_GRAPHS
    d = jnp.where(valid, d, 0.0)
    o_ref[...] = (jnp.sum(d * d) / (5.0 * NUM_GRAPHS)).reshape(1, 1)


def _head(h_pad, y_pad, p):
    rows = h_pad.shape[0]
    return pl.pallas_call(
        _head_body,
        in_specs=[
            pl.BlockSpec((rows, 2 * MD), lambda: (0, 0)),
            pl.BlockSpec((rows, 1), lambda: (0, 0)),
            pl.BlockSpec((2 * MD, MD), lambda: (0, 0)),
            pl.BlockSpec((1, MD), lambda: (0, 0)),
            pl.BlockSpec((MD, MD // 2), lambda: (0, 0)),
            pl.BlockSpec((1, MD // 2), lambda: (0, 0)),
            pl.BlockSpec((MD // 2, 1), lambda: (0, 0)),
            pl.BlockSpec((1, 1), lambda: (0, 0)),
        ],
        out_specs=pl.BlockSpec((1, 1), lambda: (0, 0)),
        out_shape=jax.ShapeDtypeStruct((1, 1), jnp.float32),
    )(
        h_pad, y_pad,
        p['mlp_w0'], p['mlp_b0'].reshape(1, MD),
        p['mlp_w1'], p['mlp_b1'].reshape(1, MD // 2),
        p['mlp_w2'], p['mlp_b2'].reshape(1, 1),
    )


@functools.lru_cache(maxsize=1)
def _make_edge_pass():
    mesh = plsc.VectorSubcoreMesh(core_axis_name="c", subcore_axis_name="s")

    @functools.partial(
        pl.kernel,
        out_type=jax.ShapeDtypeStruct((2, N_PAD, 2 * MD), jnp.float32),
        mesh=mesh,
        scratch_types=[
            pltpu.VMEM((G, K), jnp.int32),
            pltpu.VMEM((G, K), jnp.int32),
            pltpu.VMEM((2, K, 2 * MD), jnp.float32),
            pltpu.VMEM((2, K // 2, 2 * MD), jnp.float32),
            pltpu.VMEM_SHARED((N_PAD, 2 * MD), jnp.float32),
            pltpu.SemaphoreType.DMA,
            pltpu.SemaphoreType.DMA,
            pltpu.SemaphoreType.DMA,
        ],
    )
    def _edge_pass_k(nfw_hbm, e_hbm, src_hbm, dst_hbm, zeros_hbm, out_hbm,
                     src_v, dst_v, g_v, e_v, agg_sh, gsem, esem, ssem):
        c = lax.axis_index("c")
        s = lax.axis_index("s")
        wid = s * NC + c

        row0 = pl.multiple_of(s * RPS, 8)
        pltpu.sync_copy(zeros_hbm.at[pl.ds(row0, RPS)],
                        agg_sh.at[pl.ds(row0, RPS)])
        plsc.subcore_barrier()

        def _issue(grp, cg, b):
            pltpu.async_copy(nfw_hbm.at[src_v.at[cg]], g_v.at[b], gsem)
            pltpu.async_copy(e_hbm.at[wid, grp, cg], e_v.at[b], esem)

        def _wait_in(grp, cg, b):
            pltpu.make_async_copy(
                nfw_hbm.at[src_v.at[cg]], g_v.at[b], gsem).wait()
            pltpu.make_async_copy(
                e_hbm.at[wid, grp, cg], e_v.at[b], esem).wait()

        def _wait_scat(cg, b):
            pltpu.make_async_copy(
                g_v.at[b], agg_sh.at[dst_v.at[cg]], ssem).wait()

        def _step(grp, cg, b):
            @pl.when(cg >= 1)
            def _():
                _wait_scat(cg - 1, 1 - b)

            @pl.when(cg + 1 < G)
            def _():
                _issue(grp, cg + 1, 1 - b)

            _wait_in(grp, cg, b)

            @pl.loop(0, K // 2)
            def _pair(prow):
                for half in range(2):
                    r = 2 * prow + half
                    for cc in range(MD // 16):
                        g_v[b, r, pl.ds(cc * 16, 16)] = jnp.maximum(
                            g_v[b, r, pl.ds(cc * 16, 16)]
                            + e_v[b, prow, pl.ds(half * MD + cc * 16, 16)],
                            0.0)

            pltpu.async_copy(g_v.at[b], agg_sh.at[dst_v.at[cg]], ssem, add=True)

        for grp in range(NGRP):
            pltpu.sync_copy(src_hbm.at[wid, grp], src_v)
            pltpu.sync_copy(dst_hbm.at[wid, grp], dst_v)
            _issue(grp, 0, 0)

            @pl.loop(0, G - 1, step=2)
            def _chunk(cg0):
                _step(grp, cg0, 0)
                _step(grp, cg0 + 1, 1)

            _step(grp, G - 1, (G - 1) % 2)
            _wait_scat(G - 1, (G - 1) % 2)

        plsc.subcore_barrier()
        pltpu.sync_copy(agg_sh.at[pl.ds(row0, RPS)],
                        out_hbm.at[c, pl.ds(row0, RPS)])

    return _edge_pass_k


def kernel(x, edge_index, edge_attr, y, params):
    p = params
    ei = edge_index.astype(jnp.int32)
    src3 = ei[0].reshape(NW, NGRP, G, K)
    dst3 = ei[1].reshape(NW, NGRP, G, K)
    zeros = jnp.zeros((N_PAD, 2 * MD), jnp.float32)
    ea_pair = edge_attr.reshape(N_EDGES // 2, 2 * E_ENC)

    mwcat = jnp.concatenate([p['msg_w_%d' % i] for i in range(NUM_LAYERS)], axis=1)
    mbcat = jnp.concatenate([p['msg_b_%d' % i] for i in range(NUM_LAYERS)])

    nf = _mm_bias(x, p['node_enc_w'], p['node_enc_b'], block_rows=2000)
    e_terms = _edge_terms(ea_pair, p['edge_enc_w'], mwcat, p['edge_enc_b'], mbcat)
    e_terms = [e.reshape(NW, NGRP, G, K // 2, 2 * MD) for e in e_terms]

    zeros_b = jnp.zeros((MD,), jnp.float32)
    nfw = _mm_bias(nf, p['msg_w_0'], zeros_b, block_rows=2000,
                   out_rows=N_PAD, pad_cols=True)

    for i in range(NUM_LAYERS):
        agg2 = _make_edge_pass()(nfw, e_terms[i], src3, dst3, zeros)
        wt = p['upd_w_%d' % i][:MD]
        wb = p['upd_w_%d' % i][MD:]
        mw_next = p['msg_w_%d' % (i + 1)] if i + 1 < NUM_LAYERS else None
        res = _update(nf, agg2, wt, wb, p['upd_b_%d' % i], mw_next)
        if mw_next is None:
            nf = res
        else:
            nf, nfw = res

    nfr = nf.reshape(NUM_GRAPHS, NODES_PER_GRAPH, MD)
    h = jnp.concatenate([nfr[:, -10:-5, :], nfr[:, -5:, :]], axis=-1)
    h = h.reshape(5 * NUM_GRAPHS, 2 * MD)
    h_pad = jnp.pad(h, ((0, 12), (0, 0)))
    y_pad = jnp.pad(y, (0, 12)).reshape(512, 1)
    loss = _head(h_pad, y_pad, p)
    return loss.reshape(())


# speedup vs baseline: 4.2526x; 1.1287x over previous
import functools

import jax
import jax.numpy as jnp
from jax import lax
from jax.experimental import pallas as pl
from jax.experimental.pallas import tpu as pltpu
from jax.experimental.pallas import tpu_sc as plsc

N_NODES = 10000
N_EDGES = 320000
N_ENC = 128
E_ENC = 16
MD = 64
NUM_LAYERS = 3
NUM_GRAPHS = 100
NODES_PER_GRAPH = 100

NC = 2
NS = 16
NW = NC * NS
EDGES_PER_W = N_EDGES // NW
K = 80
NCHUNK = EDGES_PER_W // K
G = 25
NGRP = NCHUNK // G
N_PAD = 10240
RPS = N_PAD // NS


def _mm_bias_body(a_ref, w_ref, b_ref, o_ref):
    o_ref[...] = (
        jnp.dot(a_ref[...], w_ref[...], preferred_element_type=jnp.float32)
        + b_ref[...]
    )


def _mm_bias_pad_body(a_ref, w_ref, b_ref, o_ref):
    r = (jnp.dot(a_ref[...], w_ref[...], preferred_element_type=jnp.float32)
         + b_ref[...])
    o_ref[...] = jnp.concatenate([r, jnp.zeros_like(r)], axis=1)


def _mm_bias(a, w, b, block_rows, out_rows=None, pad_cols=False):
    rows, kdim = a.shape
    n = w.shape[1]
    out_n = 2 * n if pad_cols else n
    return pl.pallas_call(
        _mm_bias_pad_body if pad_cols else _mm_bias_body,
        grid=(rows // block_rows,),
        in_specs=[
            pl.BlockSpec((block_rows, kdim), lambda i: (i, 0)),
            pl.BlockSpec((kdim, n), lambda i: (0, 0)),
            pl.BlockSpec((1, n), lambda i: (0, 0)),
        ],
        out_specs=pl.BlockSpec((block_rows, out_n), lambda i: (i, 0)),
        out_shape=jax.ShapeDtypeStruct((out_rows or rows, out_n), jnp.float32),
    )(a, w, b.reshape(1, n))


def _eall_body(eap_ref, ew_ref, mw_ref, eb_ref, mb_ref, o0_ref, o1_ref, o2_ref):
    w2 = jnp.dot(ew_ref[...], mw_ref[...], preferred_element_type=jnp.float32)
    d = jnp.dot(eb_ref[...], mw_ref[...], preferred_element_type=jnp.float32) + mb_ref[...]
    ae = eap_ref[:, :E_ENC]
    ao = eap_ref[:, E_ENC:]
    ee = jnp.dot(ae, w2, preferred_element_type=jnp.float32) + d
    eo = jnp.dot(ao, w2, preferred_element_type=jnp.float32) + d
    for i, o_ref in enumerate((o0_ref, o1_ref, o2_ref)):
        o_ref[...] = jnp.concatenate(
            [ee[:, i * MD:(i + 1) * MD], eo[:, i * MD:(i + 1) * MD]], axis=1)


def _edge_terms(ea_pair, ew, mwcat, eb, mbcat, block_rows=2000):
    n_pair = N_EDGES // 2
    spec = pl.BlockSpec((block_rows, 2 * MD), lambda i: (i, 0))
    return pl.pallas_call(
        _eall_body,
        grid=(n_pair // block_rows,),
        in_specs=[
            pl.BlockSpec((block_rows, 2 * E_ENC), lambda i: (i, 0)),
            pl.BlockSpec((E_ENC, MD), lambda i: (0, 0)),
            pl.BlockSpec((MD, 3 * MD), lambda i: (0, 0)),
            pl.BlockSpec((1, MD), lambda i: (0, 0)),
            pl.BlockSpec((1, 3 * MD), lambda i: (0, 0)),
        ],
        out_specs=(spec, spec, spec),
        out_shape=tuple(
            jax.ShapeDtypeStruct((n_pair, 2 * MD), jnp.float32) for _ in range(3)
        ),
    )(ea_pair, ew, mwcat, eb.reshape(1, MD), mbcat.reshape(1, 3 * MD))


def _upd_body(nf_ref, a0_ref, a1_ref, wt_ref, wb_ref, b_ref, mw_ref, nfo_ref, nfwo_ref):
    agg = a0_ref[0, :, :MD] + a1_ref[0, :, :MD]
    h = (
        jnp.dot(nf_ref[...], wt_ref[...], preferred_element_type=jnp.float32)
        + jnp.dot(agg, wb_ref[...], preferred_element_type=jnp.float32)
        + b_ref[...]
    )
    nfn = nf_ref[...] + jnp.maximum(h, 0.0)
    nfo_ref[...] = nfn
    r = jnp.dot(nfn, mw_ref[...], preferred_element_type=jnp.float32)
    nfwo_ref[...] = jnp.concatenate([r, jnp.zeros_like(r)], axis=1)


def _upd_last_body(nf_ref, a0_ref, a1_ref, wt_ref, wb_ref, b_ref, nfo_ref):
    agg = a0_ref[0, :, :MD] + a1_ref[0, :, :MD]
    h = (
        jnp.dot(nf_ref[...], wt_ref[...], preferred_element_type=jnp.float32)
        + jnp.dot(agg, wb_ref[...], preferred_element_type=jnp.float32)
        + b_ref[...]
    )
    nfo_ref[...] = nf_ref[...] + jnp.maximum(h, 0.0)


def _update(nf, agg2, wt, wb, b, mw_next, block_rows=2000):
    nspec = pl.BlockSpec((block_rows, MD), lambda i: (i, 0))
    a0spec = pl.BlockSpec((1, block_rows, 2 * MD), lambda i: (0, i, 0))
    a1spec = pl.BlockSpec((1, block_rows, 2 * MD), lambda i: (1, i, 0))
    wspec = pl.BlockSpec((MD, MD), lambda i: (0, 0))
    bspec = pl.BlockSpec((1, MD), lambda i: (0, 0))
    grid = (N_NODES // block_rows,)
    if mw_next is None:
        return pl.pallas_call(
            _upd_last_body,
            grid=grid,
            in_specs=[nspec, a0spec, a1spec, wspec, wspec, bspec],
            out_specs=nspec,
            out_shape=jax.ShapeDtypeStruct((N_NODES, MD), jnp.float32),
        )(nf, agg2, agg2, wt, wb, b.reshape(1, MD))
    return pl.pallas_call(
        _upd_body,
        grid=grid,
        in_specs=[nspec, a0spec, a1spec, wspec, wspec, bspec, wspec],
        out_specs=(nspec, pl.BlockSpec((block_rows, 2 * MD), lambda i: (i, 0))),
        out_shape=(
            jax.ShapeDtypeStruct((N_NODES, MD), jnp.float32),
            jax.ShapeDtypeStruct((N_PAD, 2 * MD), jnp.float32),
        ),
    )(nf, agg2, agg2, wt, wb, b.reshape(1, MD), mw_next)


def _head_body(h_ref, y_ref, w0_ref, b0_ref, w1_ref, b1_ref, w2_ref, b2_ref, o_ref):
    h1 = jnp.maximum(
        jnp.dot(h_ref[...], w0_ref[...], preferred_element_type=jnp.float32) + b0_ref[...], 0.0)
    h2 = jnp.maximum(
        jnp.dot(h1, w1_ref[...], preferred_element_type=jnp.float32) + b1_ref[...], 0.0)
    yh = jnp.dot(h2, w2_ref[...], preferred_element_type=jnp.float32) + b2_ref[...]
    d = yh - y_ref[...]
    valid = lax.broadcasted_iota(jnp.int32, d.shape, 0) < 5 * NUM_GRAPHS
    d = jnp.where(valid, d, 0.0)
    o_ref[...] = (jnp.sum(d * d) / (5.0 * NUM_GRAPHS)).reshape(1, 1)


def _head(h_pad, y_pad, p):
    rows = h_pad.shape[0]
    return pl.pallas_call(
        _head_body,
        in_specs=[
            pl.BlockSpec((rows, 2 * MD), lambda: (0, 0)),
            pl.BlockSpec((rows, 1), lambda: (0, 0)),
            pl.BlockSpec((2 * MD, MD), lambda: (0, 0)),
            pl.BlockSpec((1, MD), lambda: (0, 0)),
            pl.BlockSpec((MD, MD // 2), lambda: (0, 0)),
            pl.BlockSpec((1, MD // 2), lambda: (0, 0)),
            pl.BlockSpec((MD // 2, 1), lambda: (0, 0)),
            pl.BlockSpec((1, 1), lambda: (0, 0)),
        ],
        out_specs=pl.BlockSpec((1, 1), lambda: (0, 0)),
        out_shape=jax.ShapeDtypeStruct((1, 1), jnp.float32),
    )(
        h_pad, y_pad,
        p['mlp_w0'], p['mlp_b0'].reshape(1, MD),
        p['mlp_w1'], p['mlp_b1'].reshape(1, MD // 2),
        p['mlp_w2'], p['mlp_b2'].reshape(1, 1),
    )


@functools.lru_cache(maxsize=1)
def _make_edge_pass():
    mesh = plsc.VectorSubcoreMesh(core_axis_name="c", subcore_axis_name="s")

    @functools.partial(
        pl.kernel,
        out_type=jax.ShapeDtypeStruct((2, N_PAD, 2 * MD), jnp.float32),
        mesh=mesh,
        scratch_types=[
            pltpu.VMEM((G, K), jnp.int32),
            pltpu.VMEM((G, K), jnp.int32),
            pltpu.VMEM((3, K, 2 * MD), jnp.float32),
            pltpu.VMEM((2, K // 2, 2 * MD), jnp.float32),
            pltpu.VMEM_SHARED((N_PAD, 2 * MD), jnp.float32),
            pltpu.SemaphoreType.DMA,
            pltpu.SemaphoreType.DMA,
            pltpu.SemaphoreType.DMA,
        ],
    )
    def _edge_pass_k(nfw_hbm, e_hbm, src_hbm, dst_hbm, zeros_hbm, out_hbm,
                     src_v, dst_v, g_v, e_v, agg_sh, gsem, esem, ssem):
        c = lax.axis_index("c")
        s = lax.axis_index("s")
        wid = s * NC + c

        row0 = pl.multiple_of(s * RPS, 8)
        pltpu.sync_copy(zeros_hbm.at[pl.ds(row0, RPS)],
                        agg_sh.at[pl.ds(row0, RPS)])
        plsc.subcore_barrier()

        def _issue(grp, cg, bg, be):
            pltpu.async_copy(nfw_hbm.at[src_v.at[cg]], g_v.at[bg], gsem)
            pltpu.async_copy(e_hbm.at[wid, grp, cg], e_v.at[be], esem)

        def _wait_in(grp, cg, bg, be):
            pltpu.make_async_copy(
                nfw_hbm.at[src_v.at[cg]], g_v.at[bg], gsem).wait()
            pltpu.make_async_copy(
                e_hbm.at[wid, grp, cg], e_v.at[be], esem).wait()

        def _wait_scat(cg, bg):
            pltpu.make_async_copy(
                g_v.at[bg], agg_sh.at[dst_v.at[cg]], ssem).wait()

        def _step(grp, cg, bg, be):
            @pl.when(cg >= 2)
            def _():
                _wait_scat(cg - 2, (bg + 1) % 3)

            @pl.when(cg + 1 < G)
            def _():
                _issue(grp, cg + 1, (bg + 1) % 3, 1 - be)

            _wait_in(grp, cg, bg, be)

            @pl.loop(0, K // 2)
            def _pair(prow):
                for half in range(2):
                    r = 2 * prow + half
                    for cc in range(MD // 16):
                        g_v[bg, r, pl.ds(cc * 16, 16)] = jnp.maximum(
                            g_v[bg, r, pl.ds(cc * 16, 16)]
                            + e_v[be, prow, pl.ds(half * MD + cc * 16, 16)],
                            0.0)

            pltpu.async_copy(g_v.at[bg], agg_sh.at[dst_v.at[cg]], ssem, add=True)

        for grp in range(NGRP):
            pltpu.sync_copy(src_hbm.at[wid, grp], src_v)
            pltpu.sync_copy(dst_hbm.at[wid, grp], dst_v)
            _issue(grp, 0, 0, 0)

            @pl.loop(0, G - 1, step=6)
            def _chunk(cg0):
                for k in range(6):
                    _step(grp, cg0 + k, k % 3, k % 2)

            _step(grp, G - 1, (G - 1) % 3, (G - 1) % 2)
            _wait_scat(G - 2, (G - 2) % 3)
            _wait_scat(G - 1, (G - 1) % 3)

        plsc.subcore_barrier()
        pltpu.sync_copy(agg_sh.at[pl.ds(row0, RPS)],
                        out_hbm.at[c, pl.ds(row0, RPS)])

    return _edge_pass_k


def kernel(x, edge_index, edge_attr, y, params):
    p = params
    ei = edge_index.astype(jnp.int32)
    src3 = ei[0].reshape(NW, NGRP, G, K)
    dst3 = ei[1].reshape(NW, NGRP, G, K)
    zeros = jnp.zeros((N_PAD, 2 * MD), jnp.float32)
    ea_pair = edge_attr.reshape(N_EDGES // 2, 2 * E_ENC)

    mwcat = jnp.concatenate([p['msg_w_%d' % i] for i in range(NUM_LAYERS)], axis=1)
    mbcat = jnp.concatenate([p['msg_b_%d' % i] for i in range(NUM_LAYERS)])

    nf = _mm_bias(x, p['node_enc_w'], p['node_enc_b'], block_rows=2000)
    e_terms = _edge_terms(ea_pair, p['edge_enc_w'], mwcat, p['edge_enc_b'], mbcat)
    e_terms = [e.reshape(NW, NGRP, G, K // 2, 2 * MD) for e in e_terms]

    zeros_b = jnp.zeros((MD,), jnp.float32)
    nfw = _mm_bias(nf, p['msg_w_0'], zeros_b, block_rows=2000,
                   out_rows=N_PAD, pad_cols=True)

    for i in range(NUM_LAYERS):
        agg2 = _make_edge_pass()(nfw, e_terms[i], src3, dst3, zeros)
        wt = p['upd_w_%d' % i][:MD]
        wb = p['upd_w_%d' % i][MD:]
        mw_next = p['msg_w_%d' % (i + 1)] if i + 1 < NUM_LAYERS else None
        res = _update(nf, agg2, wt, wb, p['upd_b_%d' % i], mw_next)
        if mw_next is None:
            nf = res
        else:
            nf, nfw = res

    nfr = nf.reshape(NUM_GRAPHS, NODES_PER_GRAPH, MD)
    h = jnp.concatenate([nfr[:, -10:-5, :], nfr[:, -5:, :]], axis=-1)
    h = h.reshape(5 * NUM_GRAPHS, 2 * MD)
    h_pad = jnp.pad(h, ((0, 12), (0, 0)))
    y_pad = jnp.pad(y, (0, 12)).reshape(512, 1)
    loss = _head(h_pad, y_pad, p)
    return loss.reshape(())


# speedup vs baseline: 4.5842x; 1.0780x over previous
import functools

import jax
import jax.numpy as jnp
from jax import lax
from jax.experimental import pallas as pl
from jax.experimental.pallas import tpu as pltpu
from jax.experimental.pallas import tpu_sc as plsc

N_NODES = 10000
N_EDGES = 320000
N_ENC = 128
E_ENC = 16
MD = 64
NUM_LAYERS = 3
NUM_GRAPHS = 100
NODES_PER_GRAPH = 100

NC = 2
NS = 16
NW = NC * NS
EDGES_PER_W = N_EDGES // NW
K = 80
NCHUNK = EDGES_PER_W // K
G = 25
NGRP = NCHUNK // G
N_PAD = 10240
RPS = N_PAD // NS


def _mm_bias_body(a_ref, w_ref, b_ref, o_ref):
    o_ref[...] = (
        jnp.dot(a_ref[...], w_ref[...], preferred_element_type=jnp.float32)
        + b_ref[...]
    )


def _mm_bias_pad_body(a_ref, w_ref, b_ref, o_ref):
    r = (jnp.dot(a_ref[...], w_ref[...], preferred_element_type=jnp.float32)
         + b_ref[...])
    o_ref[...] = jnp.concatenate([r, jnp.zeros_like(r)], axis=1)


def _mm_bias(a, w, b, block_rows, out_rows=None, pad_cols=False):
    rows, kdim = a.shape
    n = w.shape[1]
    out_n = 2 * n if pad_cols else n
    return pl.pallas_call(
        _mm_bias_pad_body if pad_cols else _mm_bias_body,
        grid=(rows // block_rows,),
        in_specs=[
            pl.BlockSpec((block_rows, kdim), lambda i: (i, 0)),
            pl.BlockSpec((kdim, n), lambda i: (0, 0)),
            pl.BlockSpec((1, n), lambda i: (0, 0)),
        ],
        out_specs=pl.BlockSpec((block_rows, out_n), lambda i: (i, 0)),
        out_shape=jax.ShapeDtypeStruct((out_rows or rows, out_n), jnp.float32),
    )(a, w, b.reshape(1, n))


def _pack_bf16_pair(lo, hi):
    lob = lax.bitcast_convert_type(lo.astype(jnp.bfloat16), jnp.uint16)
    hib = lax.bitcast_convert_type(hi.astype(jnp.bfloat16), jnp.uint16)
    word = (hib.astype(jnp.uint32) << 16) | lob.astype(jnp.uint32)
    return lax.bitcast_convert_type(word, jnp.int32)


def _eall_body(eap_ref, ew_ref, mw_ref, eb_ref, mb_ref, o0_ref, o1_ref, o2_ref):
    w2 = jnp.dot(ew_ref[...], mw_ref[...], preferred_element_type=jnp.float32)
    d = jnp.dot(eb_ref[...], mw_ref[...], preferred_element_type=jnp.float32) + mb_ref[...]
    es = [
        jnp.dot(eap_ref[:, 16 * s:16 * (s + 1)], w2,
                preferred_element_type=jnp.float32) + d
        for s in range(4)
    ]
    for i, o_ref in enumerate((o0_ref, o1_ref, o2_ref)):
        words = [
            _pack_bf16_pair(e[:, i * MD:i * MD + 32],
                            e[:, i * MD + 32:(i + 1) * MD])
            for e in es
        ]
        o_ref[...] = jnp.concatenate(words, axis=1)


def _edge_terms(ea_quad, ew, mwcat_rho, eb, mbcat_rho, block_rows=1000):
    n_quad = N_EDGES // 4
    spec = pl.BlockSpec((block_rows, 2 * MD), lambda i: (i, 0))
    return pl.pallas_call(
        _eall_body,
        grid=(n_quad // block_rows,),
        in_specs=[
            pl.BlockSpec((block_rows, 4 * E_ENC), lambda i: (i, 0)),
            pl.BlockSpec((E_ENC, MD), lambda i: (0, 0)),
            pl.BlockSpec((MD, 3 * MD), lambda i: (0, 0)),
            pl.BlockSpec((1, MD), lambda i: (0, 0)),
            pl.BlockSpec((1, 3 * MD), lambda i: (0, 0)),
        ],
        out_specs=(spec, spec, spec),
        out_shape=tuple(
            jax.ShapeDtypeStruct((n_quad, 2 * MD), jnp.int32) for _ in range(3)
        ),
    )(ea_quad, ew, mwcat_rho, eb.reshape(1, MD), mbcat_rho.reshape(1, 3 * MD))


def _upd_body(nf_ref, a0_ref, a1_ref, wt_ref, wb_ref, b_ref, mw_ref, nfo_ref, nfwo_ref):
    agg = a0_ref[0, :, :MD] + a1_ref[0, :, :MD]
    h = (
        jnp.dot(nf_ref[...], wt_ref[...], preferred_element_type=jnp.float32)
        + jnp.dot(agg, wb_ref[...], preferred_element_type=jnp.float32)
        + b_ref[...]
    )
    nfn = nf_ref[...] + jnp.maximum(h, 0.0)
    nfo_ref[...] = nfn
    r = jnp.dot(nfn, mw_ref[...], preferred_element_type=jnp.float32)
    nfwo_ref[...] = jnp.concatenate([r, jnp.zeros_like(r)], axis=1)


def _upd_last_body(nf_ref, a0_ref, a1_ref, wt_ref, wb_ref, b_ref, nfo_ref):
    agg = a0_ref[0, :, :MD] + a1_ref[0, :, :MD]
    h = (
        jnp.dot(nf_ref[...], wt_ref[...], preferred_element_type=jnp.float32)
        + jnp.dot(agg, wb_ref[...], preferred_element_type=jnp.float32)
        + b_ref[...]
    )
    nfo_ref[...] = nf_ref[...] + jnp.maximum(h, 0.0)


def _update(nf, agg2, wt, wb, b, mw_next, block_rows=2000):
    nspec = pl.BlockSpec((block_rows, MD), lambda i: (i, 0))
    a0spec = pl.BlockSpec((1, block_rows, 2 * MD), lambda i: (0, i, 0))
    a1spec = pl.BlockSpec((1, block_rows, 2 * MD), lambda i: (1, i, 0))
    wspec = pl.BlockSpec((MD, MD), lambda i: (0, 0))
    bspec = pl.BlockSpec((1, MD), lambda i: (0, 0))
    grid = (N_NODES // block_rows,)
    if mw_next is None:
        return pl.pallas_call(
            _upd_last_body,
            grid=grid,
            in_specs=[nspec, a0spec, a1spec, wspec, wspec, bspec],
            out_specs=nspec,
            out_shape=jax.ShapeDtypeStruct((N_NODES, MD), jnp.float32),
        )(nf, agg2, agg2, wt, wb, b.reshape(1, MD))
    return pl.pallas_call(
        _upd_body,
        grid=grid,
        in_specs=[nspec, a0spec, a1spec, wspec, wspec, bspec, wspec],
        out_specs=(nspec, pl.BlockSpec((block_rows, 2 * MD), lambda i: (i, 0))),
        out_shape=(
            jax.ShapeDtypeStruct((N_NODES, MD), jnp.float32),
            jax.ShapeDtypeStruct((N_PAD, 2 * MD), jnp.float32),
        ),
    )(nf, agg2, agg2, wt, wb, b.reshape(1, MD), mw_next)


def _head_body(h_ref, y_ref, w0_ref, b0_ref, w1_ref, b1_ref, w2_ref, b2_ref, o_ref):
    h1 = jnp.maximum(
        jnp.dot(h_ref[...], w0_ref[...], preferred_element_type=jnp.float32) + b0_ref[...], 0.0)
    h2 = jnp.maximum(
        jnp.dot(h1, w1_ref[...], preferred_element_type=jnp.float32) + b1_ref[...], 0.0)
    yh = jnp.dot(h2, w2_ref[...], preferred_element_type=jnp.float32) + b2_ref[...]
    d = yh - y_ref[...]
    valid = lax.broadcasted_iota(jnp.int32, d.shape, 0) < 5 * NUM_GRAPHS
    d = jnp.where(valid, d, 0.0)
    o_ref[...] = (jnp.sum(d * d) / (5.0 * NUM_GRAPHS)).reshape(1, 1)


def _head(h_pad, y_pad, p):
    rows = h_pad.shape[0]
    return pl.pallas_call(
        _head_body,
        in_specs=[
            pl.BlockSpec((rows, 2 * MD), lambda: (0, 0)),
            pl.BlockSpec((rows, 1), lambda: (0, 0)),
            pl.BlockSpec((2 * MD, MD), lambda: (0, 0)),
            pl.BlockSpec((1, MD), lambda: (0, 0)),
            pl.BlockSpec((MD, MD // 2), lambda: (0, 0)),
            pl.BlockSpec((1, MD // 2), lambda: (0, 0)),
            pl.BlockSpec((MD // 2, 1), lambda: (0, 0)),
            pl.BlockSpec((1, 1), lambda: (0, 0)),
        ],
        out_specs=pl.BlockSpec((1, 1), lambda: (0, 0)),
        out_shape=jax.ShapeDtypeStruct((1, 1), jnp.float32),
    )(
        h_pad, y_pad,
        p['mlp_w0'], p['mlp_b0'].reshape(1, MD),
        p['mlp_w1'], p['mlp_b1'].reshape(1, MD // 2),
        p['mlp_w2'], p['mlp_b2'].reshape(1, 1),
    )


@functools.lru_cache(maxsize=1)
def _make_edge_pass():
    mesh = plsc.VectorSubcoreMesh(core_axis_name="c", subcore_axis_name="s")

    @functools.partial(
        pl.kernel,
        out_type=jax.ShapeDtypeStruct((2, N_PAD, 2 * MD), jnp.float32),
        mesh=mesh,
        scratch_types=[
            pltpu.VMEM((G, K), jnp.int32),
            pltpu.VMEM((G, K), jnp.int32),
            pltpu.VMEM((3, K, 2 * MD), jnp.float32),
            pltpu.VMEM((2, K // 4, 2 * MD), jnp.int32),
            pltpu.VMEM_SHARED((N_PAD, 2 * MD), jnp.float32),
            pltpu.SemaphoreType.DMA,
            pltpu.SemaphoreType.DMA,
            pltpu.SemaphoreType.DMA,
        ],
    )
    def _edge_pass_k(nfw_hbm, e_hbm, src_hbm, dst_hbm, zeros_hbm, out_hbm,
                     src_v, dst_v, g_v, e_v, agg_sh, gsem, esem, ssem):
        c = lax.axis_index("c")
        s = lax.axis_index("s")
        wid = s * NC + c

        row0 = pl.multiple_of(s * RPS, 8)
        pltpu.sync_copy(zeros_hbm.at[pl.ds(row0, RPS)],
                        agg_sh.at[pl.ds(row0, RPS)])
        plsc.subcore_barrier()

        def _issue(grp, cg, bg, be):
            pltpu.async_copy(nfw_hbm.at[src_v.at[cg]], g_v.at[bg], gsem)
            pltpu.async_copy(e_hbm.at[wid, grp, cg], e_v.at[be], esem)

        def _wait_in(grp, cg, bg, be):
            pltpu.make_async_copy(
                nfw_hbm.at[src_v.at[cg]], g_v.at[bg], gsem).wait()
            pltpu.make_async_copy(
                e_hbm.at[wid, grp, cg], e_v.at[be], esem).wait()

        def _wait_scat(cg, bg):
            pltpu.make_async_copy(
                g_v.at[bg], agg_sh.at[dst_v.at[cg]], ssem).wait()

        def _step(grp, cg, bg, be):
            @pl.when(cg >= 2)
            def _():
                _wait_scat(cg - 2, (bg + 1) % 3)

            @pl.when(cg + 1 < G)
            def _():
                _issue(grp, cg + 1, (bg + 1) % 3, 1 - be)

            _wait_in(grp, cg, bg, be)

            @pl.loop(0, K // 4)
            def _quad(q):
                for slot in range(4):
                    r = 4 * q + slot
                    for cc in range(2):
                        ew = e_v[be, q, pl.ds(32 * slot + 16 * cc, 16)]
                        elo = lax.bitcast_convert_type(
                            lax.shift_left(ew, 16), jnp.float32)
                        ehi = lax.bitcast_convert_type(
                            ew & jnp.int32(-65536), jnp.float32)
                        slo = (bg, r, pl.ds(16 * cc, 16))
                        shi = (bg, r, pl.ds(32 + 16 * cc, 16))
                        g_v[slo] = jnp.maximum(g_v[slo] + elo, 0.0)
                        g_v[shi] = jnp.maximum(g_v[shi] + ehi, 0.0)

            pltpu.async_copy(g_v.at[bg], agg_sh.at[dst_v.at[cg]], ssem, add=True)

        for grp in range(NGRP):
            pltpu.sync_copy(src_hbm.at[wid, grp], src_v)
            pltpu.sync_copy(dst_hbm.at[wid, grp], dst_v)
            _issue(grp, 0, 0, 0)

            @pl.loop(0, G - 1, step=6)
            def _chunk(cg0):
                for k in range(6):
                    _step(grp, cg0 + k, k % 3, k % 2)

            _step(grp, G - 1, (G - 1) % 3, (G - 1) % 2)
            _wait_scat(G - 2, (G - 2) % 3)
            _wait_scat(G - 1, (G - 1) % 3)

        plsc.subcore_barrier()
        pltpu.sync_copy(agg_sh.at[pl.ds(row0, RPS)],
                        out_hbm.at[c, pl.ds(row0, RPS)])

    return _edge_pass_k


def kernel(x, edge_index, edge_attr, y, params):
    p = params
    ei = edge_index.astype(jnp.int32)
    src3 = ei[0].reshape(NW, NGRP, G, K)
    dst3 = ei[1].reshape(NW, NGRP, G, K)
    zeros = jnp.zeros((N_PAD, 2 * MD), jnp.float32)
    ea_quad = edge_attr.reshape(N_EDGES // 4, 4 * E_ENC)

    mwcat = jnp.concatenate([p['msg_w_%d' % i] for i in range(NUM_LAYERS)], axis=1)
    mbcat = jnp.concatenate([p['msg_b_%d' % i] for i in range(NUM_LAYERS)])
    tau = jnp.concatenate([jnp.arange(0, MD, 2), jnp.arange(1, MD, 2)])
    rho = jnp.concatenate([i * MD + tau for i in range(NUM_LAYERS)])

    nf = _mm_bias(x, p['node_enc_w'], p['node_enc_b'], block_rows=2000)
    e_terms = _edge_terms(ea_quad, p['edge_enc_w'], mwcat[:, rho],
                          p['edge_enc_b'], mbcat[rho])
    e_terms = [e.reshape(NW, NGRP, G, K // 4, 2 * MD) for e in e_terms]

    zeros_b = jnp.zeros((MD,), jnp.float32)
    nfw = _mm_bias(nf, p['msg_w_0'][:, tau], zeros_b, block_rows=2000,
                   out_rows=N_PAD, pad_cols=True)

    for i in range(NUM_LAYERS):
        agg2 = _make_edge_pass()(nfw, e_terms[i], src3, dst3, zeros)
        wt = p['upd_w_%d' % i][:MD]
        wb = p['upd_w_%d' % i][MD:][tau]
        mw_next = (p['msg_w_%d' % (i + 1)][:, tau]
                   if i + 1 < NUM_LAYERS else None)
        res = _update(nf, agg2, wt, wb, p['upd_b_%d' % i], mw_next)
        if mw_next is None:
            nf = res
        else:
            nf, nfw = res

    nfr = nf.reshape(NUM_GRAPHS, NODES_PER_GRAPH, MD)
    h = jnp.concatenate([nfr[:, -10:-5, :], nfr[:, -5:, :]], axis=-1)
    h = h.reshape(5 * NUM_GRAPHS, 2 * MD)
    h_pad = jnp.pad(h, ((0, 12), (0, 0)))
    y_pad = jnp.pad(y, (0, 12)).reshape(512, 1)
    loss = _head(h_pad, y_pad, p)
    return loss.reshape(())


# speedup vs baseline: 4.5981x; 1.0030x over previous
import functools

import jax
import jax.numpy as jnp
from jax import lax
from jax.experimental import pallas as pl
from jax.experimental.pallas import tpu as pltpu
from jax.experimental.pallas import tpu_sc as plsc

N_NODES = 10000
N_EDGES = 320000
N_ENC = 128
E_ENC = 16
MD = 64
NUM_LAYERS = 3
NUM_GRAPHS = 100
NODES_PER_GRAPH = 100

NC = 2
NS = 16
NW = NC * NS
EDGES_PER_W = N_EDGES // NW
K = 80
NCHUNK = EDGES_PER_W // K
G = 25
NGRP = NCHUNK // G
N_PAD = 10240
RPS = N_PAD // NS


def _mm_bias_body(a_ref, w_ref, b_ref, o_ref):
    o_ref[...] = (
        jnp.dot(a_ref[...], w_ref[...], preferred_element_type=jnp.float32)
        + b_ref[...]
    )


def _enc_body(x_ref, wn_ref, bn_ref, mw_ref, nf_ref, nfw_ref):
    nf = (jnp.dot(x_ref[...], wn_ref[...], preferred_element_type=jnp.float32)
          + bn_ref[...])
    nf_ref[...] = nf
    r = jnp.dot(nf, mw_ref[...], preferred_element_type=jnp.float32)
    nfw_ref[...] = jnp.concatenate([r, jnp.zeros_like(r)], axis=1)


def _node_enc(x, wn, bn, mw_tau, block_rows=2000):
    return pl.pallas_call(
        _enc_body,
        grid=(N_NODES // block_rows,),
        in_specs=[
            pl.BlockSpec((block_rows, N_ENC), lambda i: (i, 0)),
            pl.BlockSpec((N_ENC, MD), lambda i: (0, 0)),
            pl.BlockSpec((1, MD), lambda i: (0, 0)),
            pl.BlockSpec((MD, MD), lambda i: (0, 0)),
        ],
        out_specs=(
            pl.BlockSpec((block_rows, MD), lambda i: (i, 0)),
            pl.BlockSpec((block_rows, 2 * MD), lambda i: (i, 0)),
        ),
        out_shape=(
            jax.ShapeDtypeStruct((N_NODES, MD), jnp.float32),
            jax.ShapeDtypeStruct((N_PAD, 2 * MD), jnp.float32),
        ),
    )(x, wn, bn.reshape(1, MD), mw_tau)


def _mm_bias_pad_body(a_ref, w_ref, b_ref, o_ref):
    r = (jnp.dot(a_ref[...], w_ref[...], preferred_element_type=jnp.float32)
         + b_ref[...])
    o_ref[...] = jnp.concatenate([r, jnp.zeros_like(r)], axis=1)


def _mm_bias(a, w, b, block_rows, out_rows=None, pad_cols=False):
    rows, kdim = a.shape
    n = w.shape[1]
    out_n = 2 * n if pad_cols else n
    return pl.pallas_call(
        _mm_bias_pad_body if pad_cols else _mm_bias_body,
        grid=(rows // block_rows,),
        in_specs=[
            pl.BlockSpec((block_rows, kdim), lambda i: (i, 0)),
            pl.BlockSpec((kdim, n), lambda i: (0, 0)),
            pl.BlockSpec((1, n), lambda i: (0, 0)),
        ],
        out_specs=pl.BlockSpec((block_rows, out_n), lambda i: (i, 0)),
        out_shape=jax.ShapeDtypeStruct((out_rows or rows, out_n), jnp.float32),
    )(a, w, b.reshape(1, n))


def _pack_bf16_pair(lo, hi):
    lob = lax.bitcast_convert_type(lo.astype(jnp.bfloat16), jnp.uint16)
    hib = lax.bitcast_convert_type(hi.astype(jnp.bfloat16), jnp.uint16)
    word = (hib.astype(jnp.uint32) << 16) | lob.astype(jnp.uint32)
    return lax.bitcast_convert_type(word, jnp.int32)


def _eall_body(eap_ref, ew_ref, mw_ref, eb_ref, mb_ref, o0_ref, o1_ref, o2_ref):
    w2 = jnp.dot(ew_ref[...], mw_ref[...], preferred_element_type=jnp.float32)
    d = jnp.dot(eb_ref[...], mw_ref[...], preferred_element_type=jnp.float32) + mb_ref[...]
    es = [
        jnp.dot(eap_ref[:, 16 * s:16 * (s + 1)], w2,
                preferred_element_type=jnp.float32) + d
        for s in range(4)
    ]
    for i, o_ref in enumerate((o0_ref, o1_ref, o2_ref)):
        words = [
            _pack_bf16_pair(e[:, i * MD:i * MD + 32],
                            e[:, i * MD + 32:(i + 1) * MD])
            for e in es
        ]
        o_ref[...] = jnp.concatenate(words, axis=1)


def _edge_terms(ea_quad, ew, mwcat_rho, eb, mbcat_rho, block_rows=1000):
    n_quad = N_EDGES // 4
    spec = pl.BlockSpec((block_rows, 2 * MD), lambda i: (i, 0))
    return pl.pallas_call(
        _eall_body,
        grid=(n_quad // block_rows,),
        in_specs=[
            pl.BlockSpec((block_rows, 4 * E_ENC), lambda i: (i, 0)),
            pl.BlockSpec((E_ENC, MD), lambda i: (0, 0)),
            pl.BlockSpec((MD, 3 * MD), lambda i: (0, 0)),
            pl.BlockSpec((1, MD), lambda i: (0, 0)),
            pl.BlockSpec((1, 3 * MD), lambda i: (0, 0)),
        ],
        out_specs=(spec, spec, spec),
        out_shape=tuple(
            jax.ShapeDtypeStruct((n_quad, 2 * MD), jnp.int32) for _ in range(3)
        ),
    )(ea_quad, ew, mwcat_rho, eb.reshape(1, MD), mbcat_rho.reshape(1, 3 * MD))


def _upd_body(nf_ref, a0_ref, a1_ref, wt_ref, wb_ref, b_ref, mw_ref, nfo_ref, nfwo_ref):
    agg = a0_ref[0, :, :MD] + a1_ref[0, :, :MD]
    h = (
        jnp.dot(nf_ref[...], wt_ref[...], preferred_element_type=jnp.float32)
        + jnp.dot(agg, wb_ref[...], preferred_element_type=jnp.float32)
        + b_ref[...]
    )
    nfn = nf_ref[...] + jnp.maximum(h, 0.0)
    nfo_ref[...] = nfn
    r = jnp.dot(nfn, mw_ref[...], preferred_element_type=jnp.float32)
    nfwo_ref[...] = jnp.concatenate([r, jnp.zeros_like(r)], axis=1)


def _upd_last_body(nf_ref, a0_ref, a1_ref, wt_ref, wb_ref, b_ref, nfo_ref):
    agg = a0_ref[0, :, :MD] + a1_ref[0, :, :MD]
    h = (
        jnp.dot(nf_ref[...], wt_ref[...], preferred_element_type=jnp.float32)
        + jnp.dot(agg, wb_ref[...], preferred_element_type=jnp.float32)
        + b_ref[...]
    )
    nfo_ref[...] = nf_ref[...] + jnp.maximum(h, 0.0)


def _update(nf, agg2, wt, wb, b, mw_next, block_rows=2000):
    nspec = pl.BlockSpec((block_rows, MD), lambda i: (i, 0))
    a0spec = pl.BlockSpec((1, block_rows, 2 * MD), lambda i: (0, i, 0))
    a1spec = pl.BlockSpec((1, block_rows, 2 * MD), lambda i: (1, i, 0))
    wspec = pl.BlockSpec((MD, MD), lambda i: (0, 0))
    bspec = pl.BlockSpec((1, MD), lambda i: (0, 0))
    grid = (N_NODES // block_rows,)
    if mw_next is None:
        return pl.pallas_call(
            _upd_last_body,
            grid=grid,
            in_specs=[nspec, a0spec, a1spec, wspec, wspec, bspec],
            out_specs=nspec,
            out_shape=jax.ShapeDtypeStruct((N_NODES, MD), jnp.float32),
        )(nf, agg2, agg2, wt, wb, b.reshape(1, MD))
    return pl.pallas_call(
        _upd_body,
        grid=grid,
        in_specs=[nspec, a0spec, a1spec, wspec, wspec, bspec, wspec],
        out_specs=(nspec, pl.BlockSpec((block_rows, 2 * MD), lambda i: (i, 0))),
        out_shape=(
            jax.ShapeDtypeStruct((N_NODES, MD), jnp.float32),
            jax.ShapeDtypeStruct((N_PAD, 2 * MD), jnp.float32),
        ),
    )(nf, agg2, agg2, wt, wb, b.reshape(1, MD), mw_next)


def _head_body(h_ref, y_ref, w0_ref, b0_ref, w1_ref, b1_ref, w2_ref, b2_ref, o_ref):
    h1 = jnp.maximum(
        jnp.dot(h_ref[...], w0_ref[...], preferred_element_type=jnp.float32) + b0_ref[...], 0.0)
    h2 = jnp.maximum(
        jnp.dot(h1, w1_ref[...], preferred_element_type=jnp.float32) + b1_ref[...], 0.0)
    yh = jnp.dot(h2, w2_ref[...], preferred_element_type=jnp.float32) + b2_ref[...]
    d = yh - y_ref[...]
    valid = lax.broadcasted_iota(jnp.int32, d.shape, 0) < 5 * NUM_GRAPHS
    d = jnp.where(valid, d, 0.0)
    o_ref[...] = (jnp.sum(d * d) / (5.0 * NUM_GRAPHS)).reshape(1, 1)


def _head(h_pad, y_pad, p):
    rows = h_pad.shape[0]
    return pl.pallas_call(
        _head_body,
        in_specs=[
            pl.BlockSpec((rows, 2 * MD), lambda: (0, 0)),
            pl.BlockSpec((rows, 1), lambda: (0, 0)),
            pl.BlockSpec((2 * MD, MD), lambda: (0, 0)),
            pl.BlockSpec((1, MD), lambda: (0, 0)),
            pl.BlockSpec((MD, MD // 2), lambda: (0, 0)),
            pl.BlockSpec((1, MD // 2), lambda: (0, 0)),
            pl.BlockSpec((MD // 2, 1), lambda: (0, 0)),
            pl.BlockSpec((1, 1), lambda: (0, 0)),
        ],
        out_specs=pl.BlockSpec((1, 1), lambda: (0, 0)),
        out_shape=jax.ShapeDtypeStruct((1, 1), jnp.float32),
    )(
        h_pad, y_pad,
        p['mlp_w0'], p['mlp_b0'].reshape(1, MD),
        p['mlp_w1'], p['mlp_b1'].reshape(1, MD // 2),
        p['mlp_w2'], p['mlp_b2'].reshape(1, 1),
    )


@functools.lru_cache(maxsize=1)
def _make_edge_pass():
    mesh = plsc.VectorSubcoreMesh(core_axis_name="c", subcore_axis_name="s")

    @functools.partial(
        pl.kernel,
        out_type=jax.ShapeDtypeStruct((2, N_PAD, 2 * MD), jnp.float32),
        mesh=mesh,
        scratch_types=[
            pltpu.VMEM((G, K), jnp.int32),
            pltpu.VMEM((G, K), jnp.int32),
            pltpu.VMEM((3, K, 2 * MD), jnp.float32),
            pltpu.VMEM((2, K // 4, 2 * MD), jnp.int32),
            pltpu.VMEM_SHARED((N_PAD, 2 * MD), jnp.float32),
            pltpu.SemaphoreType.DMA,
            pltpu.SemaphoreType.DMA,
            pltpu.SemaphoreType.DMA,
        ],
    )
    def _edge_pass_k(nfw_hbm, e_hbm, src_hbm, dst_hbm, zeros_hbm, out_hbm,
                     src_v, dst_v, g_v, e_v, agg_sh, gsem, esem, ssem):
        c = lax.axis_index("c")
        s = lax.axis_index("s")
        wid = s * NC + c

        row0 = pl.multiple_of(s * RPS, 8)
        pltpu.sync_copy(zeros_hbm.at[pl.ds(row0, RPS)],
                        agg_sh.at[pl.ds(row0, RPS)])
        plsc.subcore_barrier()

        def _issue(grp, cg, bg, be):
            pltpu.async_copy(nfw_hbm.at[src_v.at[cg]], g_v.at[bg], gsem)
            pltpu.async_copy(e_hbm.at[wid, grp, cg], e_v.at[be], esem)

        def _wait_in(grp, cg, bg, be):
            pltpu.make_async_copy(
                nfw_hbm.at[src_v.at[cg]], g_v.at[bg], gsem).wait()
            pltpu.make_async_copy(
                e_hbm.at[wid, grp, cg], e_v.at[be], esem).wait()

        def _wait_scat(cg, bg):
            pltpu.make_async_copy(
                g_v.at[bg], agg_sh.at[dst_v.at[cg]], ssem).wait()

        def _step(grp, cg, bg, be):
            @pl.when(cg >= 2)
            def _():
                _wait_scat(cg - 2, (bg + 1) % 3)

            @pl.when(cg + 1 < G)
            def _():
                _issue(grp, cg + 1, (bg + 1) % 3, 1 - be)

            _wait_in(grp, cg, bg, be)

            @pl.loop(0, K // 4)
            def _quad(q):
                for slot in range(4):
                    r = 4 * q + slot
                    for cc in range(2):
                        ew = e_v[be, q, pl.ds(32 * slot + 16 * cc, 16)]
                        elo = lax.bitcast_convert_type(
                            lax.shift_left(ew, 16), jnp.float32)
                        ehi = lax.bitcast_convert_type(
                            ew & jnp.int32(-65536), jnp.float32)
                        slo = (bg, r, pl.ds(16 * cc, 16))
                        shi = (bg, r, pl.ds(32 + 16 * cc, 16))
                        g_v[slo] = jnp.maximum(g_v[slo] + elo, 0.0)
                        g_v[shi] = jnp.maximum(g_v[shi] + ehi, 0.0)

            pltpu.async_copy(g_v.at[bg], agg_sh.at[dst_v.at[cg]], ssem, add=True)

        for grp in range(NGRP):
            pltpu.sync_copy(src_hbm.at[wid, grp], src_v)
            pltpu.sync_copy(dst_hbm.at[wid, grp], dst_v)
            _issue(grp, 0, 0, 0)

            @pl.loop(0, G - 1, step=6)
            def _chunk(cg0):
                for k in range(6):
                    _step(grp, cg0 + k, k % 3, k % 2)

            _step(grp, G - 1, (G - 1) % 3, (G - 1) % 2)
            _wait_scat(G - 2, (G - 2) % 3)
            _wait_scat(G - 1, (G - 1) % 3)

        plsc.subcore_barrier()
        pltpu.sync_copy(agg_sh.at[pl.ds(row0, RPS)],
                        out_hbm.at[c, pl.ds(row0, RPS)])

    return _edge_pass_k


def kernel(x, edge_index, edge_attr, y, params):
    p = params
    ei = edge_index.astype(jnp.int32)
    src3 = ei[0].reshape(NW, NGRP, G, K)
    dst3 = ei[1].reshape(NW, NGRP, G, K)
    zeros = jnp.zeros((N_PAD, 2 * MD), jnp.float32)
    ea_quad = edge_attr.reshape(N_EDGES // 4, 4 * E_ENC)

    mwcat = jnp.concatenate([p['msg_w_%d' % i] for i in range(NUM_LAYERS)], axis=1)
    mbcat = jnp.concatenate([p['msg_b_%d' % i] for i in range(NUM_LAYERS)])
    tau = jnp.concatenate([jnp.arange(0, MD, 2), jnp.arange(1, MD, 2)])
    rho = jnp.concatenate([i * MD + tau for i in range(NUM_LAYERS)])

    nf, nfw = _node_enc(x, p['node_enc_w'], p['node_enc_b'],
                        p['msg_w_0'][:, tau])
    e_terms = _edge_terms(ea_quad, p['edge_enc_w'], mwcat[:, rho],
                          p['edge_enc_b'], mbcat[rho])
    e_terms = [e.reshape(NW, NGRP, G, K // 4, 2 * MD) for e in e_terms]

    for i in range(NUM_LAYERS):
        agg2 = _make_edge_pass()(nfw, e_terms[i], src3, dst3, zeros)
        wt = p['upd_w_%d' % i][:MD]
        wb = p['upd_w_%d' % i][MD:][tau]
        mw_next = (p['msg_w_%d' % (i + 1)][:, tau]
                   if i + 1 < NUM_LAYERS else None)
        res = _update(nf, agg2, wt, wb, p['upd_b_%d' % i], mw_next)
        if mw_next is None:
            nf = res
        else:
            nf, nfw = res

    nfr = nf.reshape(NUM_GRAPHS, NODES_PER_GRAPH, MD)
    h = jnp.concatenate([nfr[:, -10:-5, :], nfr[:, -5:, :]], axis=-1)
    h = h.reshape(5 * NUM_GRAPHS, 2 * MD)
    h_pad = jnp.pad(h, ((0, 12), (0, 0)))
    y_pad = jnp.pad(y, (0, 12)).reshape(512, 1)
    loss = _head(h_pad, y_pad, p)
    return loss.reshape(())
